# baseline (speedup 1.0000x reference)
import functools

import jax
import jax.numpy as jnp
import numpy as np
from jax import lax
from jax.experimental import pallas as pl
from jax.experimental.pallas import tpu as pltpu
from jax.experimental.pallas import tpu_sc as plsc

VOCAB = 100000
D = 64
NS = 26
B = 4096
NF = NS + 1

NW = 32
ROWS = B * NS
RPW = ROWS // NW
CHUNK = 128
NCH = RPW // CHUNK


NBUF = 6
DEPTH = 3


def _sc_gather(tables_lines, idx_flat):
  mesh = plsc.VectorSubcoreMesh(core_axis_name="c", subcore_axis_name="s")

  @functools.partial(
      pl.kernel,
      mesh=mesh,
      out_type=jax.ShapeDtypeStruct((ROWS, 2 * D), jnp.float32),
      scratch_types=[
          pltpu.VMEM((RPW,), jnp.int32),
          pltpu.VMEM((NBUF, CHUNK, 2 * D), jnp.float32),
          pltpu.SemaphoreType.DMA,
          pltpu.SemaphoreType.DMA,
      ],
  )
  def gather_k(tab_hbm, idx_hbm, out_hbm, idx_v, rows_v, gsem, wsem):
    wid = lax.axis_index("s") * 2 + lax.axis_index("c")
    pltpu.sync_copy(idx_hbm.at[pl.ds(wid * RPW, RPW)], idx_v)

    def fire(c):
      return pltpu.async_copy(
          tab_hbm.at[idx_v.at[pl.ds(c * CHUNK, CHUNK)]],
          rows_v.at[c % NBUF], gsem)

    gh = {c: fire(c) for c in range(DEPTH)}
    wh = {}
    unwaited = set()
    for c in range(NCH):
      gh.pop(c).wait()
      wh[c] = pltpu.async_copy(
          rows_v.at[c % NBUF],
          out_hbm.at[pl.ds(wid * RPW + c * CHUNK, CHUNK)], wsem)
      unwaited.add(c)
      nxt = c + DEPTH
      if nxt < NCH:
        prev_w = nxt - NBUF
        if prev_w >= 0:
          wh[prev_w].wait()
          unwaited.discard(prev_w)
        gh[nxt] = fire(nxt)
    for c in sorted(unwaited):
      wh[c].wait()

  return gather_k(tables_lines, idx_flat)


CKL = 2048
NCK = 25
LINES = CKL * NCK
VPAD = 101248


def _repack_body(tabT_ref, out_ref):
  c = pl.program_id(1)
  lo_off = pl.multiple_of(c * (2 * CKL), 128)
  hi_off = pl.multiple_of(c * (2 * CKL) + jnp.where(c < NCK - 1, CKL, 896), 128)
  lo = tabT_ref[0, :, pl.ds(lo_off, CKL)]
  hi = tabT_ref[0, :, pl.ds(hi_off, CKL)]
  xcat = jnp.concatenate([lo, hi], axis=0)
  eye = (jax.lax.broadcasted_iota(jnp.int32, (2 * D, 2 * D), 0) ==
         jax.lax.broadcasted_iota(jnp.int32, (2 * D, 2 * D), 1)
         ).astype(jnp.float32)
  out_ref[0] = jax.lax.dot_general(
      xcat, eye, (((0,), (0,)), ((), ())),
      preferred_element_type=jnp.float32)


def _repack_lines(tables):
  tt = jnp.transpose(tables, (0, 2, 1))
  return pl.pallas_call(
      _repack_body,
      grid=(NS, NCK),
      in_specs=[pl.BlockSpec((1, D, VPAD), lambda f, c: (f, 0, 0))],
      out_specs=pl.BlockSpec((1, CKL, 2 * D), lambda f, c: (f, c, 0)),
      out_shape=jax.ShapeDtypeStruct((NS, LINES, 2 * D), jnp.float32),
      compiler_params=pltpu.CompilerParams(
          vmem_limit_bytes=64 * 1024 * 1024),
  )(tt)


BT = 512
GRID = B // BT


def _tc_body(dense_ref, emb_ref, par_ref, bW0, bb0, bW1, bb1, bW2, bb2,
             tW0d, Wg3, tb0, tW1, tb1, tW2, tb2, out_ref):
  relu = lambda v: jnp.maximum(v, 0.0)
  x = dense_ref[...]
  h = relu(jnp.dot(x, bW0[...], preferred_element_type=jnp.float32) + bb0[...])
  h = relu(jnp.dot(h, bW1[...], preferred_element_type=jnp.float32) + bb1[...])
  dout = relu(jnp.dot(h, bW2[...], preferred_element_type=jnp.float32) + bb2[...])

  lines = emb_ref[...]
  par = par_ref[...]
  emb3 = jnp.where(par > 0.0, lines[:, :, D:], lines[:, :, :D])
  f3 = jnp.concatenate(
      [dout.reshape(BT, 1, D), emb3], axis=1).astype(jnp.bfloat16)
  g = lax.dot_general(f3, f3, (((2,), (2,)), ((0,), (0,))),
                      preferred_element_type=jnp.float32)
  gb = g.astype(jnp.bfloat16)

  z = jnp.dot(dout, tW0d[...], preferred_element_type=jnp.float32) + tb0[...]
  for n in range(NF):
    z = z + jnp.dot(gb[:, n, :], Wg3[n].astype(jnp.bfloat16),
                    preferred_element_type=jnp.float32)
  y = relu(z)
  y = relu(jnp.dot(y, tW1[...], preferred_element_type=jnp.float32) + tb1[...])
  y = relu(jnp.dot(y, tW2[...], preferred_element_type=jnp.float32) + tb2[...])
  out_ref[...] = y


def _tc_call(dense, emb3, par3, bW0, bb0, bW1, bb1, bW2, bb2,
             tW0d, Wg3, tb0, tW1, tb1, tW2, tb2):
  full2 = lambda shape: pl.BlockSpec(shape, lambda i: (0, 0))
  return pl.pallas_call(
      _tc_body,
      grid=(GRID,),
      in_specs=[
          pl.BlockSpec((BT, 13), lambda i: (i, 0)),
          pl.BlockSpec((BT, NS, 2 * D), lambda i: (i, 0, 0)),
          pl.BlockSpec((BT, NS, 1), lambda i: (i, 0, 0)),
          full2((13, 512)), full2((1, 512)),
          full2((512, 256)), full2((1, 256)),
          full2((256, 64)), full2((1, 64)),
          full2((64, 512)),
          pl.BlockSpec((NF, NF, 512), lambda i: (0, 0, 0)),
          full2((1, 512)),
          full2((512, 256)), full2((1, 256)),
          full2((256, 1)), full2((1, 1)),
      ],
      out_specs=pl.BlockSpec((BT, 1), lambda i: (i, 0)),
      out_shape=jax.ShapeDtypeStruct((B, 1), jnp.float32),
  )(dense, emb3, par3, bW0, bb0, bW1, bb1, bW2, bb2,
    tW0d, Wg3, tb0, tW1, tb1, tW2, tb2)


def kernel(dense, sparse, tables, bW0, bb0, bW1, bb1, bW2, bb2,
           tW0, tb0, tW1, tb1, tW2, tb2):
  sparse = sparse.astype(jnp.int32)
  k = sparse >> 12
  jj = sparse & 4095
  ck = jnp.where(k < NCK - 1, CKL, 896)
  in_hi = jj >= ck
  line = k * CKL + jj - jnp.where(in_hi, ck, 0)
  offs = (jnp.arange(NS, dtype=jnp.int32) * LINES)[None, :]
  idx_flat = (line + offs).reshape(ROWS)
  par3 = in_hi.astype(jnp.float32).reshape(B, NS, 1)
  tables_lines = _repack_lines(tables).reshape(NS * LINES, 2 * D)
  emb = _sc_gather(tables_lines, idx_flat)
  emb3 = emb.reshape(B, NS, 2 * D)

  tri0, tri1 = np.triu_indices(NF, k=1)
  tW0d = tW0[:D]
  Wg3 = jnp.zeros((NF, NF, 512), jnp.float32).at[tri0, tri1].set(tW0[D:])

  out = _tc_call(
      dense, emb3, par3, bW0, bb0.reshape(1, -1), bW1, bb1.reshape(1, -1),
      bW2, bb2.reshape(1, -1), tW0d, Wg3, tb0.reshape(1, -1),
      tW1, tb1.reshape(1, -1), tW2, tb2.reshape(1, -1))
  return out[:, 0]


# speedup vs baseline: 1.1539x; 1.1539x over previous
import functools

import jax
import jax.numpy as jnp
import numpy as np
from jax import lax
from jax.experimental import pallas as pl
from jax.experimental.pallas import tpu as pltpu
from jax.experimental.pallas import tpu_sc as plsc

VOCAB = 100000
D = 64
NS = 26
B = 4096
NF = NS + 1

NW = 32
ROWS = B * NS
RPW = ROWS // NW
CHUNK = 128
NCH = RPW // CHUNK


NBUF = 6
DEPTH = 3


def _sc_gather(tables_lines, idx_flat):
  mesh = plsc.VectorSubcoreMesh(core_axis_name="c", subcore_axis_name="s")

  @functools.partial(
      pl.kernel,
      mesh=mesh,
      out_type=jax.ShapeDtypeStruct((ROWS, 2 * D), jnp.float32),
      scratch_types=[
          pltpu.VMEM((RPW,), jnp.int32),
          pltpu.VMEM((NBUF, CHUNK, 2 * D), jnp.float32),
          pltpu.SemaphoreType.DMA,
          pltpu.SemaphoreType.DMA,
      ],
  )
  def gather_k(tab_hbm, idx_hbm, out_hbm, idx_v, rows_v, gsem, wsem):
    wid = lax.axis_index("s") * 2 + lax.axis_index("c")
    pltpu.sync_copy(idx_hbm.at[pl.ds(wid * RPW, RPW)], idx_v)

    def fire(c):
      return pltpu.async_copy(
          tab_hbm.at[idx_v.at[pl.ds(c * CHUNK, CHUNK)]],
          rows_v.at[c % NBUF], gsem)

    gh = {c: fire(c) for c in range(DEPTH)}
    wh = {}
    unwaited = set()
    for c in range(NCH):
      gh.pop(c).wait()
      wh[c] = pltpu.async_copy(
          rows_v.at[c % NBUF],
          out_hbm.at[pl.ds(wid * RPW + c * CHUNK, CHUNK)], wsem)
      unwaited.add(c)
      nxt = c + DEPTH
      if nxt < NCH:
        prev_w = nxt - NBUF
        if prev_w >= 0:
          wh[prev_w].wait()
          unwaited.discard(prev_w)
        gh[nxt] = fire(nxt)
    for c in sorted(unwaited):
      wh[c].wait()

  return gather_k(tables_lines, idx_flat)


CKL = 4096
NCK = 13
TAILCK = 896
LINES = CKL * NCK
VPAD = 103296


def _repack_body(tabT_ref, out_ref):
  c = pl.program_id(1)
  lo_off = pl.multiple_of(c * (2 * CKL), 128)
  hi_off = pl.multiple_of(
      c * (2 * CKL) + jnp.where(c < NCK - 1, CKL, TAILCK), 128)
  lo = tabT_ref[0, :, pl.ds(lo_off, CKL)]
  hi = tabT_ref[0, :, pl.ds(hi_off, CKL)]
  xcat = jnp.concatenate([lo, hi], axis=0)
  eye = (jax.lax.broadcasted_iota(jnp.int32, (2 * D, 2 * D), 0) ==
         jax.lax.broadcasted_iota(jnp.int32, (2 * D, 2 * D), 1)
         ).astype(jnp.float32)
  out_ref[0] = jax.lax.dot_general(
      xcat, eye, (((0,), (0,)), ((), ())),
      preferred_element_type=jnp.float32)


def _repack_lines(tables):
  tt = jnp.transpose(tables, (0, 2, 1))
  return pl.pallas_call(
      _repack_body,
      grid=(NS, NCK),
      in_specs=[pl.BlockSpec((1, D, VPAD), lambda f, c: (f, 0, 0))],
      out_specs=pl.BlockSpec((1, CKL, 2 * D), lambda f, c: (f, c, 0)),
      out_shape=jax.ShapeDtypeStruct((NS, LINES, 2 * D), jnp.float32),
      compiler_params=pltpu.CompilerParams(
          vmem_limit_bytes=64 * 1024 * 1024),
  )(tt)


BT = 256
GRID = B // BT


def _tc_body(dense_ref, emb_ref, par_ref, bW0, bb0, bW1, bb1, bW2, bb2,
             tW0d, Wg3, tb0, tW1, tb1, tW2, tb2, out_ref):
  relu = lambda v: jnp.maximum(v, 0.0)
  x = dense_ref[...]
  h = relu(jnp.dot(x, bW0[...], preferred_element_type=jnp.float32) + bb0[...])
  h = relu(jnp.dot(h, bW1[...], preferred_element_type=jnp.float32) + bb1[...])
  dout = relu(jnp.dot(h, bW2[...], preferred_element_type=jnp.float32) + bb2[...])

  lines = emb_ref[...]
  par = par_ref[...]
  emb3 = jnp.where(par > 0.0, lines[:, :, D:], lines[:, :, :D])
  f3 = jnp.concatenate(
      [dout.reshape(BT, 1, D), emb3], axis=1).astype(jnp.bfloat16)
  g = lax.dot_general(f3, f3, (((2,), (2,)), ((0,), (0,))),
                      preferred_element_type=jnp.float32)
  gb = g.astype(jnp.bfloat16)

  z = jnp.dot(dout, tW0d[...], preferred_element_type=jnp.float32) + tb0[...]
  for n in range(NF):
    z = z + jnp.dot(gb[:, n, :], Wg3[n].astype(jnp.bfloat16),
                    preferred_element_type=jnp.float32)
  y = relu(z)
  y = relu(jnp.dot(y, tW1[...], preferred_element_type=jnp.float32) + tb1[...])
  y = relu(jnp.dot(y, tW2[...], preferred_element_type=jnp.float32) + tb2[...])
  out_ref[...] = y


def _tc_call(dense, emb3, par3, bW0, bb0, bW1, bb1, bW2, bb2,
             tW0d, Wg3, tb0, tW1, tb1, tW2, tb2):
  full2 = lambda shape: pl.BlockSpec(shape, lambda i: (0, 0))
  return pl.pallas_call(
      _tc_body,
      grid=(GRID,),
      in_specs=[
          pl.BlockSpec((BT, 13), lambda i: (i, 0)),
          pl.BlockSpec((BT, NS, 2 * D), lambda i: (i, 0, 0)),
          pl.BlockSpec((BT, NS, 1), lambda i: (i, 0, 0)),
          full2((13, 512)), full2((1, 512)),
          full2((512, 256)), full2((1, 256)),
          full2((256, 64)), full2((1, 64)),
          full2((64, 512)),
          pl.BlockSpec((NF, NF, 512), lambda i: (0, 0, 0)),
          full2((1, 512)),
          full2((512, 256)), full2((1, 256)),
          full2((256, 1)), full2((1, 1)),
      ],
      out_specs=pl.BlockSpec((BT, 1), lambda i: (i, 0)),
      out_shape=jax.ShapeDtypeStruct((B, 1), jnp.float32),
  )(dense, emb3, par3, bW0, bb0, bW1, bb1, bW2, bb2,
    tW0d, Wg3, tb0, tW1, tb1, tW2, tb2)


def kernel(dense, sparse, tables, bW0, bb0, bW1, bb1, bW2, bb2,
           tW0, tb0, tW1, tb1, tW2, tb2):
  sparse = sparse.astype(jnp.int32)
  k = sparse >> 13
  jj = sparse & 8191
  ck = jnp.where(k < NCK - 1, CKL, TAILCK)
  in_hi = jj >= ck
  line = k * CKL + jj - jnp.where(in_hi, ck, 0)
  offs = (jnp.arange(NS, dtype=jnp.int32) * LINES)[None, :]
  idx_flat = (line + offs).reshape(ROWS)
  par3 = in_hi.astype(jnp.float32).reshape(B, NS, 1)
  tables_lines = _repack_lines(tables).reshape(NS * LINES, 2 * D)
  emb = _sc_gather(tables_lines, idx_flat)
  emb3 = emb.reshape(B, NS, 2 * D)

  tri0, tri1 = np.triu_indices(NF, k=1)
  tW0d = tW0[:D]
  Wg3 = jnp.zeros((NF, NF, 512), jnp.float32).at[tri0, tri1].set(tW0[D:])

  out = _tc_call(
      dense, emb3, par3, bW0, bb0.reshape(1, -1), bW1, bb1.reshape(1, -1),
      bW2, bb2.reshape(1, -1), tW0d, Wg3, tb0.reshape(1, -1),
      tW1, tb1.reshape(1, -1), tW2, tb2.reshape(1, -1))
  return out[:, 0]
